# baseline (device time: 250409 ns/iter reference)
import numpy as np
import jax
import jax.numpy as jnp
from jax import lax
from jax.experimental import pallas as pl
from jax.experimental.pallas import tpu as pltpu

N_DEV = 4
SQ = 1024
SKV = 1024
H_LOC = 8
DH = 128
DM = 1024
BLK = 64
SCALE = 0.08838834764831843

_PBLK = [b for r in range(3) for b in range(16) if b % 3 == r]
_IPBLK = [_PBLK.index(o) for o in range(16)]
NA = 6 * BLK
NBC = 5 * BLK


def _permute_blocks(a, order, axis):
    sl = [None] * a.ndim
    chunks = []
    for b in order:
        idx = list(sl)
        idx[axis] = slice(b * BLK, (b + 1) * BLK)
        chunks.append(a[tuple(slice(None) if s is None else s for s in idx)])
    return jnp.concatenate(chunks, axis=axis)


def kernel(x, Wq, K_ext, V_ext, Wo):
    my = lax.axis_index("i")
    K_sl = lax.dynamic_slice_in_dim(K_ext, my * H_LOC, H_LOC, axis=2)
    V_sl = lax.dynamic_slice_in_dim(V_ext, my * H_LOC, H_LOC, axis=2)
    K_sl = _permute_blocks(
        K_sl.transpose(0, 2, 1, 3).astype(jnp.bfloat16), _PBLK, axis=2)
    V_sl = _permute_blocks(
        V_sl.transpose(0, 2, 1, 3).astype(jnp.bfloat16), _PBLK, axis=2)
    x_b = _permute_blocks(x.astype(jnp.bfloat16), _PBLK, axis=1)
    Wq_b = Wq.astype(jnp.bfloat16).reshape(DM, H_LOC, DH).transpose(1, 0, 2)
    Wo_b = Wo.astype(jnp.bfloat16).reshape(H_LOC, DH, DM)

    def body(x_ref, wq_ref, k_ref, v_ref, wo_ref, out_ref,
             xbuf, arecv, asend, xs, xr, as_, ar):
        my_pos = lax.axis_index("i")
        left = (my_pos + N_DEV - 1) % N_DEV
        right = (my_pos + 1) % N_DEV

        def partial_for(x_val, b):

            def head_body(h, acc):
                k = k_ref[b, h]
                v = v_ref[b, h]
                q = (lax.dot_general(
                    x_val, wq_ref[h], (((1,), (0,)), ((), ())),
                    preferred_element_type=jnp.float32,
                ) * SCALE).astype(jnp.bfloat16)

                cdims = (((1,), (1,)), ((), ()))
                s_a = lax.dot_general(
                    q[0:NA], k[0:NA], cdims,
                    preferred_element_type=jnp.float32,
                )
                w_a = jnp.exp(s_a)
                d_a = jnp.sum(w_a, axis=1, keepdims=True)
                ctx_a = lax.dot_general(
                    w_a.astype(jnp.bfloat16), v[0:NA],
                    (((1,), (0,)), ((), ())),
                    preferred_element_type=jnp.float32,
                ) / d_a

                k_b = jnp.concatenate([k[0:BLK], k[NA + NBC:]], axis=0)
                v_b = jnp.concatenate([v[0:BLK], v[NA + NBC:]], axis=0)
                s_b = lax.dot_general(
                    q[NA:NA + NBC], k_b, cdims,
                    preferred_element_type=jnp.float32,
                )
                w_b = jnp.exp(s_b)
                d_b = jnp.sum(w_b, axis=1, keepdims=True)
                ctx_b = lax.dot_general(
                    w_b.astype(jnp.bfloat16), v_b,
                    (((1,), (0,)), ((), ())),
                    preferred_element_type=jnp.float32,
                )

                k_c = jnp.concatenate([k[0:BLK], k[NA:NA + NBC]], axis=0)
                v_c = jnp.concatenate([v[0:BLK], v[NA:NA + NBC]], axis=0)
                s_c = lax.dot_general(
                    q[NA + NBC:], k_c, cdims,
                    preferred_element_type=jnp.float32,
                )
                w_c = jnp.exp(s_c)
                d_c = jnp.sum(w_c, axis=1, keepdims=True)
                ctx_c = lax.dot_general(
                    w_c.astype(jnp.bfloat16), v_c,
                    (((1,), (0,)), ((), ())),
                    preferred_element_type=jnp.float32,
                )

                q_d = q[NA:].reshape(10, BLK, DH)
                k_d = k[NA:].reshape(10, BLK, DH)
                v_d = v[NA:].reshape(10, BLK, DH)
                s_d = lax.dot_general(
                    q_d, k_d, (((2,), (2,)), ((0,), (0,))),
                    preferred_element_type=jnp.float32,
                )
                w_d = jnp.exp(s_d)
                d_d = jnp.sum(w_d, axis=2, keepdims=True)
                ctx_d = lax.dot_general(
                    w_d.astype(jnp.bfloat16), v_d,
                    (((2,), (1,)), ((0,), (0,))),
                    preferred_element_type=jnp.float32,
                ).reshape(2 * NBC, DH)
                d_d = d_d.reshape(2 * NBC, 1)

                ctx = jnp.concatenate([
                    ctx_a,
                    (ctx_b + ctx_d[0:NBC]) / (d_b + d_d[0:NBC]),
                    (ctx_c + ctx_d[NBC:]) / (d_c + d_d[NBC:]),
                ], axis=0)
                return acc + lax.dot_general(
                    ctx.astype(jnp.bfloat16), wo_ref[h],
                    (((1,), (0,)), ((), ())),
                    preferred_element_type=jnp.float32,
                )

            return lax.fori_loop(
                0, H_LOC, head_body, jnp.zeros((SQ, DM), jnp.float32)
            )

        def xcopy(src, slot):
            return pltpu.make_async_remote_copy(
                src_ref=src, dst_ref=xbuf.at[slot],
                send_sem=xs.at[slot], recv_sem=xr.at[slot],
                device_id=(right,), device_id_type=pl.DeviceIdType.MESH,
            )

        def acopy(slot):
            return pltpu.make_async_remote_copy(
                src_ref=asend, dst_ref=arecv.at[slot],
                send_sem=as_.at[slot], recv_sem=ar.at[slot],
                device_id=(right,), device_id_type=pl.DeviceIdType.MESH,
            )

        barrier = pltpu.get_barrier_semaphore()
        for nbr in (left, right):
            pl.semaphore_signal(
                barrier, inc=1,
                device_id=(nbr,), device_id_type=pl.DeviceIdType.MESH,
            )
        pl.semaphore_wait(barrier, 2)

        cx0 = xcopy(x_ref.at[0], 0)
        cx0.start()
        p_own = partial_for(x_ref[0], my_pos)

        cx0.wait()
        cx1 = xcopy(xbuf.at[0], 1)
        cx1.start()
        p = partial_for(xbuf[0], (my_pos + 3) % N_DEV)
        asend[:, :] = p.astype(jnp.bfloat16)
        ca0 = acopy(0)
        ca0.start()

        cx1.wait()
        cx2 = xcopy(xbuf.at[1], 2)
        cx2.start()
        p = partial_for(xbuf[1], (my_pos + 2) % N_DEV)
        ca0.wait()
        asend[:, :] = (arecv[0] + p).astype(jnp.bfloat16)
        ca1 = acopy(1)
        ca1.start()

        cx2.wait()
        p = partial_for(xbuf[2], (my_pos + 1) % N_DEV)
        ca1.wait()
        asend[:, :] = (arecv[1] + p).astype(jnp.bfloat16)
        ca2 = acopy(2)
        ca2.start()

        ca2.wait()
        out_ref[0, :, :] = arecv[2] + p_own

    out = pl.pallas_call(
        body,
        out_shape=jax.ShapeDtypeStruct((1, SQ, DM), jnp.float32),
        in_specs=[pl.BlockSpec(memory_space=pltpu.VMEM)] * 5,
        out_specs=pl.BlockSpec(memory_space=pltpu.VMEM),
        scratch_shapes=[
            pltpu.VMEM((3, SQ, DM), jnp.bfloat16),
            pltpu.VMEM((3, SQ, DM), jnp.bfloat16),
            pltpu.VMEM((SQ, DM), jnp.bfloat16),
            pltpu.SemaphoreType.DMA((3,)),
            pltpu.SemaphoreType.DMA((3,)),
            pltpu.SemaphoreType.DMA((3,)),
            pltpu.SemaphoreType.DMA((3,)),
        ],
        compiler_params=pltpu.CompilerParams(
            collective_id=0, vmem_limit_bytes=100 * 1024 * 1024,
        ),
    )(x_b, Wq_b, K_sl, V_sl, Wo_b)
    return _permute_blocks(out, _IPBLK, axis=1)


# device time: 191640 ns/iter; 1.3067x vs baseline; 1.3067x over previous
import numpy as np
import jax
import jax.numpy as jnp
from jax import lax
from jax.experimental import pallas as pl
from jax.experimental.pallas import tpu as pltpu

N_DEV = 4
SQ = 1024
SKV = 1024
H_LOC = 8
DH = 128
DM = 1024
SCALE = 0.08838834764831843

_qb = (np.arange(SQ) // 64)[:, None]
_kb = (np.arange(SKV) // 64)[None, :]
_mask = (_qb == _kb) | (_kb == 0) | ((_qb + _kb) % 3 == 0)
_BIAS = np.where(_mask, 0.0, -1e9).astype(np.float32)


def kernel(x, Wq, K_ext, V_ext, Wo):
    x_b = x.astype(jnp.bfloat16)
    Wq_b = Wq.astype(jnp.bfloat16).reshape(DM, H_LOC, DH).transpose(1, 0, 2)
    Wo_b = Wo.astype(jnp.bfloat16).reshape(H_LOC, DH, DM)
    bias = jnp.asarray(_BIAS, dtype=jnp.bfloat16)

    def body(x_ref, wq_ref, k_ref, v_ref, wo_ref, bias_ref, out_ref,
             xbuf, arecv, asend, kbuf, vbuf, ksem, vsem, xs, xr, as_, ar):
        my_pos = lax.axis_index("i")
        left = (my_pos + N_DEV - 1) % N_DEV
        right = (my_pos + 1) % N_DEV
        h0 = my_pos * H_LOC

        def kv_copies(b, h, slot):
            ck = pltpu.make_async_copy(
                k_ref.at[b, :, h0 + h, :], kbuf.at[slot], ksem.at[slot])
            cv = pltpu.make_async_copy(
                v_ref.at[b, :, h0 + h, :], vbuf.at[slot], vsem.at[slot])
            return ck, cv

        def partial_for(x_val, b):
            ck, cv = kv_copies(b, 0, 0)
            ck.start()
            cv.start()

            def head_body(h, acc):
                slot = lax.rem(h, 2)

                @pl.when(h + 1 < H_LOC)
                def _():
                    nck, ncv = kv_copies(b, h + 1, lax.rem(h + 1, 2))
                    nck.start()
                    ncv.start()

                ckw, cvw = kv_copies(b, h, slot)
                ckw.wait()
                cvw.wait()
                k = kbuf[slot].astype(jnp.bfloat16)
                v = vbuf[slot].astype(jnp.bfloat16)
                q = (lax.dot_general(
                    x_val, wq_ref[h], (((1,), (0,)), ((), ())),
                    preferred_element_type=jnp.float32,
                ) * SCALE).astype(jnp.bfloat16)
                s = lax.dot_general(
                    q, k, (((1,), (1,)), ((), ())),
                    preferred_element_type=jnp.float32,
                )
                w = jnp.exp(s + bias_ref[:, :])
                denom = jnp.sum(w, axis=1, keepdims=True)
                ctx = lax.dot_general(
                    w.astype(jnp.bfloat16), v,
                    (((1,), (0,)), ((), ())),
                    preferred_element_type=jnp.float32,
                ) / denom
                return acc + lax.dot_general(
                    ctx.astype(jnp.bfloat16), wo_ref[h],
                    (((1,), (0,)), ((), ())),
                    preferred_element_type=jnp.float32,
                )

            return lax.fori_loop(
                0, H_LOC, head_body, jnp.zeros((SQ, DM), jnp.float32)
            )

        def xcopy(src, slot):
            return pltpu.make_async_remote_copy(
                src_ref=src, dst_ref=xbuf.at[slot],
                send_sem=xs.at[slot], recv_sem=xr.at[slot],
                device_id=(right,), device_id_type=pl.DeviceIdType.MESH,
            )

        def acopy(slot):
            return pltpu.make_async_remote_copy(
                src_ref=asend, dst_ref=arecv.at[slot],
                send_sem=as_.at[slot], recv_sem=ar.at[slot],
                device_id=(right,), device_id_type=pl.DeviceIdType.MESH,
            )

        barrier = pltpu.get_barrier_semaphore()
        for nbr in (left, right):
            pl.semaphore_signal(
                barrier, inc=1,
                device_id=(nbr,), device_id_type=pl.DeviceIdType.MESH,
            )
        pl.semaphore_wait(barrier, 2)

        cx0 = xcopy(x_ref.at[0], 0)
        cx0.start()
        p_own = partial_for(x_ref[0], my_pos)

        cx0.wait()
        cx1 = xcopy(xbuf.at[0], 1)
        cx1.start()
        p = partial_for(xbuf[0], (my_pos + 3) % N_DEV)
        asend[:, :] = p.astype(jnp.bfloat16)
        ca0 = acopy(0)
        ca0.start()

        cx1.wait()
        cx2 = xcopy(xbuf.at[1], 2)
        cx2.start()
        p = partial_for(xbuf[1], (my_pos + 2) % N_DEV)
        ca0.wait()
        asend[:, :] = (arecv[0] + p).astype(jnp.bfloat16)
        ca1 = acopy(1)
        ca1.start()

        cx2.wait()
        p = partial_for(xbuf[2], (my_pos + 1) % N_DEV)
        ca1.wait()
        asend[:, :] = (arecv[1] + p).astype(jnp.bfloat16)
        ca2 = acopy(2)
        ca2.start()

        ca2.wait()
        out_ref[0, :, :] = arecv[2] + p_own

    out = pl.pallas_call(
        body,
        out_shape=jax.ShapeDtypeStruct((1, SQ, DM), jnp.float32),
        in_specs=[
            pl.BlockSpec(memory_space=pltpu.VMEM),
            pl.BlockSpec(memory_space=pltpu.VMEM),
            pl.BlockSpec(memory_space=pl.ANY),
            pl.BlockSpec(memory_space=pl.ANY),
            pl.BlockSpec(memory_space=pltpu.VMEM),
            pl.BlockSpec(memory_space=pltpu.VMEM),
        ],
        out_specs=pl.BlockSpec(memory_space=pltpu.VMEM),
        scratch_shapes=[
            pltpu.VMEM((3, SQ, DM), jnp.bfloat16),
            pltpu.VMEM((3, SQ, DM), jnp.bfloat16),
            pltpu.VMEM((SQ, DM), jnp.bfloat16),
            pltpu.VMEM((2, SKV, DH), jnp.float32),
            pltpu.VMEM((2, SKV, DH), jnp.float32),
            pltpu.SemaphoreType.DMA((2,)),
            pltpu.SemaphoreType.DMA((2,)),
            pltpu.SemaphoreType.DMA((3,)),
            pltpu.SemaphoreType.DMA((3,)),
            pltpu.SemaphoreType.DMA((3,)),
            pltpu.SemaphoreType.DMA((3,)),
        ],
        compiler_params=pltpu.CompilerParams(
            collective_id=0, vmem_limit_bytes=100 * 1024 * 1024,
        ),
    )(x_b, Wq_b, K_ext, V_ext, Wo_b, bias)
    return out


# device time: 188162 ns/iter; 1.3308x vs baseline; 1.0185x over previous
import jax
import jax.numpy as jnp
from jax import lax
from jax.experimental import pallas as pl
from jax.experimental.pallas import tpu as pltpu

N_DEV = 4
SQ = 1024
SKV = 1024
H_LOC = 8
DH = 128
DM = 1024
BLK = 64
NBLK = 16
SCALE = 0.08838834764831843

_PBLK = [b for r in range(3) for b in range(NBLK) if b % 3 == r]
NA = 6 * BLK
NBC = 5 * BLK


def kernel(x, Wq, K_ext, V_ext, Wo):
    x_b = x.astype(jnp.bfloat16)
    Wq_b = Wq.astype(jnp.bfloat16).reshape(DM, H_LOC, DH).transpose(1, 0, 2)
    Wo_b = Wo.astype(jnp.bfloat16).reshape(H_LOC, DH, DM)

    def body(x_ref, wq_ref, k_ref, v_ref, wo_ref, out_ref,
             xbuf, arecv, asend, xperm, kbuf, vbuf, ksem, vsem,
             xs, xr, as_, ar):
        my_pos = lax.axis_index("i")
        left = (my_pos + N_DEV - 1) % N_DEV
        right = (my_pos + 1) % N_DEV
        h0 = my_pos * H_LOC

        def kv_copies(b, h, slot):
            cps = []
            for j, blk in enumerate(_PBLK):
                cps.append(pltpu.make_async_copy(
                    k_ref.at[b, pl.ds(blk * BLK, BLK), h0 + h, :],
                    kbuf.at[slot, pl.ds(j * BLK, BLK), :],
                    ksem.at[slot, j]))
                cps.append(pltpu.make_async_copy(
                    v_ref.at[b, pl.ds(blk * BLK, BLK), h0 + h, :],
                    vbuf.at[slot, pl.ds(j * BLK, BLK), :],
                    vsem.at[slot, j]))
            return cps

        def fill_xperm(src_ref, idx):
            for j, blk in enumerate(_PBLK):
                xperm[j * BLK:(j + 1) * BLK, :] = (
                    src_ref[idx, blk * BLK:(blk + 1) * BLK, :])

        def partial_for(b):
            for c in kv_copies(b, 0, 0):
                c.start()

            def head_body(h, acc):
                slot = lax.rem(h, 2)

                @pl.when(h + 1 < H_LOC)
                def _():
                    for c in kv_copies(b, h + 1, lax.rem(h + 1, 2)):
                        c.start()

                for c in kv_copies(b, h, slot):
                    c.wait()

                q = (lax.dot_general(
                    xperm[:, :], wq_ref[h], (((1,), (0,)), ((), ())),
                    preferred_element_type=jnp.float32,
                ) * SCALE).astype(jnp.bfloat16)

                cd = (((1,), (1,)), ((), ()))
                cn = (((1,), (0,)), ((), ()))
                f32 = jnp.float32
                bf16 = jnp.bfloat16

                k_a = kbuf[slot, 0:NA, :].astype(bf16)
                v_a = vbuf[slot, 0:NA, :].astype(bf16)
                w_a = jnp.exp(lax.dot_general(
                    q[0:NA], k_a, cd, preferred_element_type=f32))
                d_a = jnp.sum(w_a, axis=1, keepdims=True)
                ctx_a = lax.dot_general(
                    w_a.astype(bf16), v_a, cn,
                    preferred_element_type=f32) / d_a

                k_0 = kbuf[slot, 0:BLK, :].astype(bf16)
                v_0 = vbuf[slot, 0:BLK, :].astype(bf16)
                k_bs = kbuf[slot, NA + NBC:, :].astype(bf16)
                v_bs = vbuf[slot, NA + NBC:, :].astype(bf16)
                k_cs = kbuf[slot, NA:NA + NBC, :].astype(bf16)
                v_cs = vbuf[slot, NA:NA + NBC, :].astype(bf16)

                q_b = q[NA:NA + NBC]
                w_b0 = jnp.exp(lax.dot_general(
                    q_b, k_0, cd, preferred_element_type=f32))
                w_b1 = jnp.exp(lax.dot_general(
                    q_b, k_bs, cd, preferred_element_type=f32))
                d_b = (jnp.sum(w_b0, axis=1, keepdims=True)
                       + jnp.sum(w_b1, axis=1, keepdims=True))
                ctx_b = (lax.dot_general(
                    w_b0.astype(bf16), v_0, cn, preferred_element_type=f32)
                    + lax.dot_general(
                    w_b1.astype(bf16), v_bs, cn, preferred_element_type=f32))

                q_c = q[NA + NBC:]
                w_c0 = jnp.exp(lax.dot_general(
                    q_c, k_0, cd, preferred_element_type=f32))
                w_c1 = jnp.exp(lax.dot_general(
                    q_c, k_cs, cd, preferred_element_type=f32))
                d_c = (jnp.sum(w_c0, axis=1, keepdims=True)
                       + jnp.sum(w_c1, axis=1, keepdims=True))
                ctx_c = (lax.dot_general(
                    w_c0.astype(bf16), v_0, cn, preferred_element_type=f32)
                    + lax.dot_general(
                    w_c1.astype(bf16), v_cs, cn, preferred_element_type=f32))

                q_d = q[NA:].reshape(10, BLK, DH)
                k_d = kbuf[slot, NA:, :].astype(bf16).reshape(10, BLK, DH)
                v_d = vbuf[slot, NA:, :].astype(bf16).reshape(10, BLK, DH)
                w_d = jnp.exp(lax.dot_general(
                    q_d, k_d, (((2,), (2,)), ((0,), (0,))),
                    preferred_element_type=f32))
                d_d = jnp.sum(w_d, axis=2, keepdims=True)
                ctx_d = lax.dot_general(
                    w_d.astype(bf16), v_d, (((2,), (1,)), ((0,), (0,))),
                    preferred_element_type=f32,
                ).reshape(2 * NBC, DH)
                d_d = d_d.reshape(2 * NBC, 1)

                ctx = jnp.concatenate([
                    ctx_a,
                    (ctx_b + ctx_d[0:NBC]) / (d_b + d_d[0:NBC]),
                    (ctx_c + ctx_d[NBC:]) / (d_c + d_d[NBC:]),
                ], axis=0)
                return acc + lax.dot_general(
                    ctx.astype(bf16), wo_ref[h], cn,
                    preferred_element_type=f32)

            return lax.fori_loop(
                0, H_LOC, head_body, jnp.zeros((SQ, DM), jnp.float32)
            )

        def xcopy(src, slot):
            return pltpu.make_async_remote_copy(
                src_ref=src, dst_ref=xbuf.at[slot],
                send_sem=xs.at[slot], recv_sem=xr.at[slot],
                device_id=(right,), device_id_type=pl.DeviceIdType.MESH,
            )

        def acopy(slot):
            return pltpu.make_async_remote_copy(
                src_ref=asend, dst_ref=arecv.at[slot],
                send_sem=as_.at[slot], recv_sem=ar.at[slot],
                device_id=(right,), device_id_type=pl.DeviceIdType.MESH,
            )

        barrier = pltpu.get_barrier_semaphore()
        for nbr in (left, right):
            pl.semaphore_signal(
                barrier, inc=1,
                device_id=(nbr,), device_id_type=pl.DeviceIdType.MESH,
            )
        pl.semaphore_wait(barrier, 2)

        cx0 = xcopy(x_ref.at[0], 0)
        cx0.start()
        fill_xperm(x_ref, 0)
        p_own = partial_for(my_pos)

        cx0.wait()
        cx1 = xcopy(xbuf.at[0], 1)
        cx1.start()
        fill_xperm(xbuf, 0)
        p = partial_for((my_pos + 3) % N_DEV)
        asend[:, :] = p.astype(jnp.bfloat16)
        ca0 = acopy(0)
        ca0.start()

        cx1.wait()
        cx2 = xcopy(xbuf.at[1], 2)
        cx2.start()
        fill_xperm(xbuf, 1)
        p = partial_for((my_pos + 2) % N_DEV)
        ca0.wait()
        asend[:, :] = (arecv[0] + p).astype(jnp.bfloat16)
        ca1 = acopy(1)
        ca1.start()

        cx2.wait()
        fill_xperm(xbuf, 2)
        p = partial_for((my_pos + 1) % N_DEV)
        ca1.wait()
        asend[:, :] = (arecv[1] + p).astype(jnp.bfloat16)
        ca2 = acopy(2)
        ca2.start()

        ca2.wait()
        total = arecv[2] + p_own
        for j, blk in enumerate(_PBLK):
            out_ref[0, blk * BLK:(blk + 1) * BLK, :] = (
                total[j * BLK:(j + 1) * BLK, :])

    out = pl.pallas_call(
        body,
        out_shape=jax.ShapeDtypeStruct((1, SQ, DM), jnp.float32),
        in_specs=[
            pl.BlockSpec(memory_space=pltpu.VMEM),
            pl.BlockSpec(memory_space=pltpu.VMEM),
            pl.BlockSpec(memory_space=pl.ANY),
            pl.BlockSpec(memory_space=pl.ANY),
            pl.BlockSpec(memory_space=pltpu.VMEM),
        ],
        out_specs=pl.BlockSpec(memory_space=pltpu.VMEM),
        scratch_shapes=[
            pltpu.VMEM((3, SQ, DM), jnp.bfloat16),
            pltpu.VMEM((3, SQ, DM), jnp.bfloat16),
            pltpu.VMEM((SQ, DM), jnp.bfloat16),
            pltpu.VMEM((SQ, DM), jnp.bfloat16),
            pltpu.VMEM((2, SKV, DH), jnp.float32),
            pltpu.VMEM((2, SKV, DH), jnp.float32),
            pltpu.SemaphoreType.DMA((2, NBLK)),
            pltpu.SemaphoreType.DMA((2, NBLK)),
            pltpu.SemaphoreType.DMA((3,)),
            pltpu.SemaphoreType.DMA((3,)),
            pltpu.SemaphoreType.DMA((3,)),
            pltpu.SemaphoreType.DMA((3,)),
        ],
        compiler_params=pltpu.CompilerParams(
            collective_id=0, vmem_limit_bytes=100 * 1024 * 1024,
        ),
    )(x_b, Wq_b, K_ext, V_ext, Wo_b)
    return out


# device time: 162425 ns/iter; 1.5417x vs baseline; 1.1585x over previous
import jax
import jax.numpy as jnp
from jax import lax
from jax.experimental import pallas as pl
from jax.experimental.pallas import tpu as pltpu

N_DEV = 4
SQ = 1024
SKV = 1024
H_LOC = 8
DH = 128
DM = 1024
BLK = 64
NBLK = 16
SCALE = 0.08838834764831843

_PBLK = [b for r in range(3) for b in range(NBLK) if b % 3 == r]
NA = 6 * BLK
NBC = 5 * BLK


def kernel(x, Wq, K_ext, V_ext, Wo):
    x_b = x.astype(jnp.bfloat16)
    Wq_b = Wq.astype(jnp.bfloat16)
    Wo_b = Wo.astype(jnp.bfloat16)

    def body(x_ref, wq_ref, k_ref, v_ref, wo_ref, out_ref,
             xbuf, arecv, asend, xperm, qall, ctxall, kbuf, vbuf,
             ksem, vsem, xs, xr, as_, ar):
        my_pos = lax.axis_index("i")
        left = (my_pos + N_DEV - 1) % N_DEV
        right = (my_pos + 1) % N_DEV
        h0 = my_pos * H_LOC

        def kv_copies(b, h, slot):
            cps = []
            for j, blk in enumerate(_PBLK):
                cps.append(pltpu.make_async_copy(
                    k_ref.at[b, pl.ds(blk * BLK, BLK), h0 + h, :],
                    kbuf.at[slot, pl.ds(j * BLK, BLK), :],
                    ksem.at[slot, j]))
                cps.append(pltpu.make_async_copy(
                    v_ref.at[b, pl.ds(blk * BLK, BLK), h0 + h, :],
                    vbuf.at[slot, pl.ds(j * BLK, BLK), :],
                    vsem.at[slot, j]))
            return cps

        def fill_xperm(src_ref, idx):
            for j, blk in enumerate(_PBLK):
                xperm[j * BLK:(j + 1) * BLK, :] = (
                    src_ref[idx, blk * BLK:(blk + 1) * BLK, :])

        def partial_for(b):
            for c in kv_copies(b, 0, 0):
                c.start()

            qall[:, :] = (lax.dot_general(
                xperm[:, :], wq_ref[:, :], (((1,), (0,)), ((), ())),
                preferred_element_type=jnp.float32,
            ) * SCALE).astype(jnp.bfloat16)

            def head_body(h, carry):
                slot = lax.rem(h, 2)

                @pl.when(h + 1 < H_LOC)
                def _():
                    for c in kv_copies(b, h + 1, lax.rem(h + 1, 2)):
                        c.start()

                for c in kv_copies(b, h, slot):
                    c.wait()

                q = qall[:, pl.ds(h * DH, DH)]

                cd = (((1,), (1,)), ((), ()))
                cn = (((1,), (0,)), ((), ()))
                f32 = jnp.float32
                bf16 = jnp.bfloat16

                k_a = kbuf[slot, 0:NA, :].astype(bf16)
                v_a = vbuf[slot, 0:NA, :].astype(bf16)
                w_a = jnp.exp(lax.dot_general(
                    q[0:NA], k_a, cd, preferred_element_type=f32))
                d_a = jnp.sum(w_a, axis=1, keepdims=True)
                ctx_a = lax.dot_general(
                    w_a.astype(bf16), v_a, cn,
                    preferred_element_type=f32) / d_a

                k_0 = kbuf[slot, 0:BLK, :].astype(bf16)
                v_0 = vbuf[slot, 0:BLK, :].astype(bf16)
                k_bs = kbuf[slot, NA + NBC:, :].astype(bf16)
                v_bs = vbuf[slot, NA + NBC:, :].astype(bf16)
                k_cs = kbuf[slot, NA:NA + NBC, :].astype(bf16)
                v_cs = vbuf[slot, NA:NA + NBC, :].astype(bf16)

                q_b = q[NA:NA + NBC]
                w_b0 = jnp.exp(lax.dot_general(
                    q_b, k_0, cd, preferred_element_type=f32))
                w_b1 = jnp.exp(lax.dot_general(
                    q_b, k_bs, cd, preferred_element_type=f32))
                d_b = (jnp.sum(w_b0, axis=1, keepdims=True)
                       + jnp.sum(w_b1, axis=1, keepdims=True))
                ctx_b = (lax.dot_general(
                    w_b0.astype(bf16), v_0, cn, preferred_element_type=f32)
                    + lax.dot_general(
                    w_b1.astype(bf16), v_bs, cn, preferred_element_type=f32))

                q_c = q[NA + NBC:]
                w_c0 = jnp.exp(lax.dot_general(
                    q_c, k_0, cd, preferred_element_type=f32))
                w_c1 = jnp.exp(lax.dot_general(
                    q_c, k_cs, cd, preferred_element_type=f32))
                d_c = (jnp.sum(w_c0, axis=1, keepdims=True)
                       + jnp.sum(w_c1, axis=1, keepdims=True))
                ctx_c = (lax.dot_general(
                    w_c0.astype(bf16), v_0, cn, preferred_element_type=f32)
                    + lax.dot_general(
                    w_c1.astype(bf16), v_cs, cn, preferred_element_type=f32))

                q_d = q[NA:].reshape(10, BLK, DH)
                k_d = kbuf[slot, NA:, :].astype(bf16).reshape(10, BLK, DH)
                v_d = vbuf[slot, NA:, :].astype(bf16).reshape(10, BLK, DH)
                w_d = jnp.exp(lax.dot_general(
                    q_d, k_d, (((2,), (2,)), ((0,), (0,))),
                    preferred_element_type=f32))
                d_d = jnp.sum(w_d, axis=2, keepdims=True)
                ctx_d = lax.dot_general(
                    w_d.astype(bf16), v_d, (((2,), (1,)), ((0,), (0,))),
                    preferred_element_type=f32,
                ).reshape(2 * NBC, DH)
                d_d = d_d.reshape(2 * NBC, 1)

                ctx = jnp.concatenate([
                    ctx_a,
                    (ctx_b + ctx_d[0:NBC]) / (d_b + d_d[0:NBC]),
                    (ctx_c + ctx_d[NBC:]) / (d_c + d_d[NBC:]),
                ], axis=0)
                ctxall[:, pl.ds(h * DH, DH)] = ctx.astype(bf16)
                return carry

            lax.fori_loop(0, H_LOC, head_body, jnp.int32(0))
            return lax.dot_general(
                ctxall[:, :], wo_ref[:, :], (((1,), (0,)), ((), ())),
                preferred_element_type=jnp.float32,
            )

        def xcopy(src, slot):
            return pltpu.make_async_remote_copy(
                src_ref=src, dst_ref=xbuf.at[slot],
                send_sem=xs.at[slot], recv_sem=xr.at[slot],
                device_id=(right,), device_id_type=pl.DeviceIdType.MESH,
            )

        def acopy(slot):
            return pltpu.make_async_remote_copy(
                src_ref=asend, dst_ref=arecv.at[slot],
                send_sem=as_.at[slot], recv_sem=ar.at[slot],
                device_id=(right,), device_id_type=pl.DeviceIdType.MESH,
            )

        barrier = pltpu.get_barrier_semaphore()
        for nbr in (left, right):
            pl.semaphore_signal(
                barrier, inc=1,
                device_id=(nbr,), device_id_type=pl.DeviceIdType.MESH,
            )
        pl.semaphore_wait(barrier, 2)

        cx0 = xcopy(x_ref.at[0], 0)
        cx0.start()
        fill_xperm(x_ref, 0)
        p_own = partial_for(my_pos)

        cx0.wait()
        cx1 = xcopy(xbuf.at[0], 1)
        cx1.start()
        fill_xperm(xbuf, 0)
        p = partial_for((my_pos + 3) % N_DEV)
        asend[:, :] = p.astype(jnp.bfloat16)
        ca0 = acopy(0)
        ca0.start()

        cx1.wait()
        cx2 = xcopy(xbuf.at[1], 2)
        cx2.start()
        fill_xperm(xbuf, 1)
        p = partial_for((my_pos + 2) % N_DEV)
        ca0.wait()
        asend[:, :] = (arecv[0] + p).astype(jnp.bfloat16)
        ca1 = acopy(1)
        ca1.start()

        cx2.wait()
        fill_xperm(xbuf, 2)
        p = partial_for((my_pos + 1) % N_DEV)
        ca1.wait()
        asend[:, :] = (arecv[1] + p).astype(jnp.bfloat16)
        ca2 = acopy(2)
        ca2.start()

        ca2.wait()
        total = arecv[2] + p_own
        for j, blk in enumerate(_PBLK):
            out_ref[0, blk * BLK:(blk + 1) * BLK, :] = (
                total[j * BLK:(j + 1) * BLK, :])

    out = pl.pallas_call(
        body,
        out_shape=jax.ShapeDtypeStruct((1, SQ, DM), jnp.float32),
        in_specs=[
            pl.BlockSpec(memory_space=pltpu.VMEM),
            pl.BlockSpec(memory_space=pltpu.VMEM),
            pl.BlockSpec(memory_space=pl.ANY),
            pl.BlockSpec(memory_space=pl.ANY),
            pl.BlockSpec(memory_space=pltpu.VMEM),
        ],
        out_specs=pl.BlockSpec(memory_space=pltpu.VMEM),
        scratch_shapes=[
            pltpu.VMEM((3, SQ, DM), jnp.bfloat16),
            pltpu.VMEM((3, SQ, DM), jnp.bfloat16),
            pltpu.VMEM((SQ, DM), jnp.bfloat16),
            pltpu.VMEM((SQ, DM), jnp.bfloat16),
            pltpu.VMEM((SQ, DM), jnp.bfloat16),
            pltpu.VMEM((SQ, DM), jnp.bfloat16),
            pltpu.VMEM((2, SKV, DH), jnp.float32),
            pltpu.VMEM((2, SKV, DH), jnp.float32),
            pltpu.SemaphoreType.DMA((2, NBLK)),
            pltpu.SemaphoreType.DMA((2, NBLK)),
            pltpu.SemaphoreType.DMA((3,)),
            pltpu.SemaphoreType.DMA((3,)),
            pltpu.SemaphoreType.DMA((3,)),
            pltpu.SemaphoreType.DMA((3,)),
        ],
        compiler_params=pltpu.CompilerParams(
            collective_id=0, vmem_limit_bytes=100 * 1024 * 1024,
        ),
    )(x_b, Wq_b, K_ext, V_ext, Wo_b)
    return out


# device time: 119619 ns/iter; 2.0934x vs baseline; 1.3579x over previous
import jax
import jax.numpy as jnp
from jax import lax
from jax.experimental import pallas as pl
from jax.experimental.pallas import tpu as pltpu

N_DEV = 4
SQ = 1024
SKV = 1024
H_LOC = 8
DH = 128
DM = 1024
BLK = 64
NBLK = 16
SCALE = 0.08838834764831843

_PBLK = [b for r in range(3) for b in range(NBLK) if b % 3 == r]
NA = 6 * BLK
NBC = 5 * BLK


def kernel(x, Wq, K_ext, V_ext, Wo):
    x_b = x.astype(jnp.bfloat16)
    Wq_b = Wq.astype(jnp.bfloat16)
    Wo_b = Wo.astype(jnp.bfloat16)

    def body(x_ref, wq_ref, k_ref, v_ref, wo_ref, out_ref,
             wqbuf, wobuf, xperm, qall, ctxall, kbuf, vbuf,
             ksem, vsem, wqs, wqr, wos, wor):
        my_pos = lax.axis_index("i")
        left = (my_pos + N_DEV - 1) % N_DEV
        right = (my_pos + 1) % N_DEV
        b = my_pos

        def kv_copies(src_dev, h, slot):
            hg = src_dev * H_LOC + h
            cps = []
            for j, blk in enumerate(_PBLK):
                cps.append(pltpu.make_async_copy(
                    k_ref.at[b, pl.ds(blk * BLK, BLK), hg, :],
                    kbuf.at[slot, pl.ds(j * BLK, BLK), :],
                    ksem.at[slot, j]))
                cps.append(pltpu.make_async_copy(
                    v_ref.at[b, pl.ds(blk * BLK, BLK), hg, :],
                    vbuf.at[slot, pl.ds(j * BLK, BLK), :],
                    vsem.at[slot, j]))
            return cps

        def compute_block(wq_chunk, src_dev, t):
            for c in kv_copies(src_dev, 0, 0):
                c.start()

            qall[:, :] = (lax.dot_general(
                xperm[:, :], wq_chunk[:, :], (((1,), (0,)), ((), ())),
                preferred_element_type=jnp.float32,
            ) * SCALE).astype(jnp.bfloat16)

            def head_body(h, carry):
                slot = lax.rem(h, 2)

                @pl.when(h + 1 < H_LOC)
                def _():
                    for c in kv_copies(src_dev, h + 1, lax.rem(h + 1, 2)):
                        c.start()

                for c in kv_copies(src_dev, h, slot):
                    c.wait()

                q = qall[:, pl.ds(h * DH, DH)]

                cd = (((1,), (1,)), ((), ()))
                cn = (((1,), (0,)), ((), ()))
                f32 = jnp.float32
                bf16 = jnp.bfloat16

                k_a = kbuf[slot, 0:NA, :].astype(bf16)
                v_a = vbuf[slot, 0:NA, :].astype(bf16)
                w_a = jnp.exp(lax.dot_general(
                    q[0:NA], k_a, cd, preferred_element_type=f32))
                d_a = jnp.sum(w_a, axis=1, keepdims=True)
                ctx_a = lax.dot_general(
                    w_a.astype(bf16), v_a, cn,
                    preferred_element_type=f32) / d_a

                k_0 = kbuf[slot, 0:BLK, :].astype(bf16)
                v_0 = vbuf[slot, 0:BLK, :].astype(bf16)
                k_bs = kbuf[slot, NA + NBC:, :].astype(bf16)
                v_bs = vbuf[slot, NA + NBC:, :].astype(bf16)
                k_cs = kbuf[slot, NA:NA + NBC, :].astype(bf16)
                v_cs = vbuf[slot, NA:NA + NBC, :].astype(bf16)

                q_b = q[NA:NA + NBC]
                w_b0 = jnp.exp(lax.dot_general(
                    q_b, k_0, cd, preferred_element_type=f32))
                w_b1 = jnp.exp(lax.dot_general(
                    q_b, k_bs, cd, preferred_element_type=f32))
                d_b = (jnp.sum(w_b0, axis=1, keepdims=True)
                       + jnp.sum(w_b1, axis=1, keepdims=True))
                ctx_b = (lax.dot_general(
                    w_b0.astype(bf16), v_0, cn, preferred_element_type=f32)
                    + lax.dot_general(
                    w_b1.astype(bf16), v_bs, cn, preferred_element_type=f32))

                q_c = q[NA + NBC:]
                w_c0 = jnp.exp(lax.dot_general(
                    q_c, k_0, cd, preferred_element_type=f32))
                w_c1 = jnp.exp(lax.dot_general(
                    q_c, k_cs, cd, preferred_element_type=f32))
                d_c = (jnp.sum(w_c0, axis=1, keepdims=True)
                       + jnp.sum(w_c1, axis=1, keepdims=True))
                ctx_c = (lax.dot_general(
                    w_c0.astype(bf16), v_0, cn, preferred_element_type=f32)
                    + lax.dot_general(
                    w_c1.astype(bf16), v_cs, cn, preferred_element_type=f32))

                q_d = q[NA:].reshape(10, BLK, DH)
                k_d = kbuf[slot, NA:, :].astype(bf16).reshape(10, BLK, DH)
                v_d = vbuf[slot, NA:, :].astype(bf16).reshape(10, BLK, DH)
                w_d = jnp.exp(lax.dot_general(
                    q_d, k_d, (((2,), (2,)), ((0,), (0,))),
                    preferred_element_type=f32))
                d_d = jnp.sum(w_d, axis=2, keepdims=True)
                ctx_d = lax.dot_general(
                    w_d.astype(bf16), v_d, (((2,), (1,)), ((0,), (0,))),
                    preferred_element_type=f32,
                ).reshape(2 * NBC, DH)
                d_d = d_d.reshape(2 * NBC, 1)

                ctx = jnp.concatenate([
                    ctx_a,
                    (ctx_b + ctx_d[0:NBC]) / (d_b + d_d[0:NBC]),
                    (ctx_c + ctx_d[NBC:]) / (d_c + d_d[NBC:]),
                ], axis=0)
                ctxall[:, pl.ds(t * DM + h * DH, DH)] = ctx.astype(bf16)
                return carry

            lax.fori_loop(0, H_LOC, head_body, jnp.int32(0))

        def wq_copy(src, slot):
            return pltpu.make_async_remote_copy(
                src_ref=src, dst_ref=wqbuf.at[slot],
                send_sem=wqs.at[slot], recv_sem=wqr.at[slot],
                device_id=(right,), device_id_type=pl.DeviceIdType.MESH,
            )

        def wo_copy(src, slot):
            return pltpu.make_async_remote_copy(
                src_ref=src, dst_ref=wobuf.at[slot],
                send_sem=wos.at[slot], recv_sem=wor.at[slot],
                device_id=(left,), device_id_type=pl.DeviceIdType.MESH,
            )

        barrier = pltpu.get_barrier_semaphore()
        for nbr in (left, right):
            pl.semaphore_signal(
                barrier, inc=1,
                device_id=(nbr,), device_id_type=pl.DeviceIdType.MESH,
            )
        pl.semaphore_wait(barrier, 2)

        cq0 = wq_copy(wq_ref, 0)
        cq0.start()
        co0 = wo_copy(wo_ref, 0)
        co0.start()

        for j, blk in enumerate(_PBLK):
            xperm[j * BLK:(j + 1) * BLK, :] = (
                x_ref[0, blk * BLK:(blk + 1) * BLK, :])

        compute_block(wq_ref, my_pos, 0)

        cq1 = cq2 = co1 = co2 = None
        for t in range(1, N_DEV):
            wq_in = wq_copy(wq_ref, t - 1)
            wq_in.wait()
            wo_in = wo_copy(wo_ref, t - 1)
            wo_in.wait()
            if t < N_DEV - 1:
                cq = wq_copy(wqbuf.at[t - 1], t)
                cq.start()
                co = wo_copy(wobuf.at[t - 1], t)
                co.start()
            compute_block(wqbuf.at[t - 1], (my_pos + N_DEV - t) % N_DEV, t)

        acc = lax.dot_general(
            ctxall[:, pl.ds(0, DM)], wo_ref[:, :], (((1,), (0,)), ((), ())),
            preferred_element_type=jnp.float32,
        )
        for t in range(1, N_DEV):
            acc = acc + lax.dot_general(
                ctxall[:, pl.ds(t * DM, DM)], wobuf[3 - t],
                (((1,), (0,)), ((), ())),
                preferred_element_type=jnp.float32,
            )

        for j, blk in enumerate(_PBLK):
            out_ref[0, blk * BLK:(blk + 1) * BLK, :] = (
                acc[j * BLK:(j + 1) * BLK, :])

    out = pl.pallas_call(
        body,
        out_shape=jax.ShapeDtypeStruct((1, SQ, DM), jnp.float32),
        in_specs=[
            pl.BlockSpec(memory_space=pltpu.VMEM),
            pl.BlockSpec(memory_space=pltpu.VMEM),
            pl.BlockSpec(memory_space=pl.ANY),
            pl.BlockSpec(memory_space=pl.ANY),
            pl.BlockSpec(memory_space=pltpu.VMEM),
        ],
        out_specs=pl.BlockSpec(memory_space=pltpu.VMEM),
        scratch_shapes=[
            pltpu.VMEM((3, DM, DM), jnp.bfloat16),
            pltpu.VMEM((3, DM, DM), jnp.bfloat16),
            pltpu.VMEM((SQ, DM), jnp.bfloat16),
            pltpu.VMEM((SQ, DM), jnp.bfloat16),
            pltpu.VMEM((SQ, N_DEV * DM), jnp.bfloat16),
            pltpu.VMEM((2, SKV, DH), jnp.float32),
            pltpu.VMEM((2, SKV, DH), jnp.float32),
            pltpu.SemaphoreType.DMA((2, NBLK)),
            pltpu.SemaphoreType.DMA((2, NBLK)),
            pltpu.SemaphoreType.DMA((3,)),
            pltpu.SemaphoreType.DMA((3,)),
            pltpu.SemaphoreType.DMA((3,)),
            pltpu.SemaphoreType.DMA((3,)),
        ],
        compiler_params=pltpu.CompilerParams(
            collective_id=0, vmem_limit_bytes=100 * 1024 * 1024,
        ),
    )(x_b, Wq_b, K_ext, V_ext, Wo_b)
    return out


# device time: 118484 ns/iter; 2.1134x vs baseline; 1.0096x over previous
import jax
import jax.numpy as jnp
from jax import lax
from jax.experimental import pallas as pl
from jax.experimental.pallas import tpu as pltpu

N_DEV = 4
SQ = 1024
SKV = 1024
H_LOC = 8
DH = 128
DM = 1024
BLK = 64
NBLK = 16
SCALE = 0.08838834764831843

_PBLK = [b for r in range(3) for b in range(NBLK) if b % 3 == r]
NA = 6 * BLK
NBC = 5 * BLK


def kernel(x, Wq, K_ext, V_ext, Wo):
    x_b = x.astype(jnp.bfloat16)
    Wq_b = Wq.astype(jnp.bfloat16)
    Wo_b = Wo.astype(jnp.bfloat16)

    def body(x_ref, wq_ref, k_ref, v_ref, wo_ref, out_ref,
             wqbuf, wobuf, xperm, qall, ctxall, wfull, kbuf, vbuf,
             ksem, vsem, wqs, wqr, wos, wor):
        my_pos = lax.axis_index("i")
        left = (my_pos + N_DEV - 1) % N_DEV
        right = (my_pos + 1) % N_DEV
        b = my_pos

        def kv_copies(src_dev, h, slot):
            hg = src_dev * H_LOC + h
            cps = []
            for j, blk in enumerate(_PBLK):
                cps.append(pltpu.make_async_copy(
                    k_ref.at[b, pl.ds(blk * BLK, BLK), hg, :],
                    kbuf.at[slot, pl.ds(j * BLK, BLK), :],
                    ksem.at[slot, j]))
                cps.append(pltpu.make_async_copy(
                    v_ref.at[b, pl.ds(blk * BLK, BLK), hg, :],
                    vbuf.at[slot, pl.ds(j * BLK, BLK), :],
                    vsem.at[slot, j]))
            return cps

        def compute_block(wq_chunk, src_dev, t, prefetched=False):
            if not prefetched:
                for c in kv_copies(src_dev, 0, 0):
                    c.start()

            qall[:, :] = (lax.dot_general(
                xperm[:, :], wq_chunk[:, :], (((1,), (0,)), ((), ())),
                preferred_element_type=jnp.float32,
            ) * SCALE).astype(jnp.bfloat16)

            def head_body(h, carry):
                slot = lax.rem(h, 2)

                @pl.when(h + 1 < H_LOC)
                def _():
                    for c in kv_copies(src_dev, h + 1, lax.rem(h + 1, 2)):
                        c.start()

                for c in kv_copies(src_dev, h, slot):
                    c.wait()

                q = qall[:, pl.ds(h * DH, DH)]

                cd = (((1,), (1,)), ((), ()))
                cn = (((1,), (0,)), ((), ()))
                f32 = jnp.float32
                bf16 = jnp.bfloat16

                kk = kbuf[slot].astype(bf16)
                vv = vbuf[slot].astype(bf16)

                k_a = kk[0:NA]
                v_a = vv[0:NA]
                w_a = jnp.exp(lax.dot_general(
                    q[0:NA], k_a, cd, preferred_element_type=f32))
                d_a = jnp.sum(w_a, axis=1, keepdims=True)
                ctx_a = lax.dot_general(
                    w_a.astype(bf16), v_a, cn,
                    preferred_element_type=f32) / d_a

                k_0 = kk[0:BLK]
                v_0 = vv[0:BLK]
                k_bs = kk[NA + NBC:]
                v_bs = vv[NA + NBC:]
                k_cs = kk[NA:NA + NBC]
                v_cs = vv[NA:NA + NBC]

                q_b = q[NA:NA + NBC]
                w_b0 = jnp.exp(lax.dot_general(
                    q_b, k_0, cd, preferred_element_type=f32))
                w_b1 = jnp.exp(lax.dot_general(
                    q_b, k_bs, cd, preferred_element_type=f32))
                d_b = (jnp.sum(w_b0, axis=1, keepdims=True)
                       + jnp.sum(w_b1, axis=1, keepdims=True))
                ctx_b = (lax.dot_general(
                    w_b0.astype(bf16), v_0, cn, preferred_element_type=f32)
                    + lax.dot_general(
                    w_b1.astype(bf16), v_bs, cn, preferred_element_type=f32))

                q_c = q[NA + NBC:]
                w_c0 = jnp.exp(lax.dot_general(
                    q_c, k_0, cd, preferred_element_type=f32))
                w_c1 = jnp.exp(lax.dot_general(
                    q_c, k_cs, cd, preferred_element_type=f32))
                d_c = (jnp.sum(w_c0, axis=1, keepdims=True)
                       + jnp.sum(w_c1, axis=1, keepdims=True))
                ctx_c = (lax.dot_general(
                    w_c0.astype(bf16), v_0, cn, preferred_element_type=f32)
                    + lax.dot_general(
                    w_c1.astype(bf16), v_cs, cn, preferred_element_type=f32))

                q_d = q[NA:].reshape(10, BLK, DH)
                k_d = kk[NA:].reshape(10, BLK, DH)
                v_d = vv[NA:].reshape(10, BLK, DH)
                w_d = jnp.exp(lax.dot_general(
                    q_d, k_d, (((2,), (2,)), ((0,), (0,))),
                    preferred_element_type=f32))
                d_d = jnp.sum(w_d, axis=2, keepdims=True)
                ctx_d = lax.dot_general(
                    w_d.astype(bf16), v_d, (((2,), (1,)), ((0,), (0,))),
                    preferred_element_type=f32,
                ).reshape(2 * NBC, DH)
                d_d = d_d.reshape(2 * NBC, 1)

                ctx = jnp.concatenate([
                    ctx_a,
                    (ctx_b + ctx_d[0:NBC]) / (d_b + d_d[0:NBC]),
                    (ctx_c + ctx_d[NBC:]) / (d_c + d_d[NBC:]),
                ], axis=0)
                ctxall[:, pl.ds(t * DM + h * DH, DH)] = ctx.astype(bf16)
                return carry

            lax.fori_loop(0, H_LOC, head_body, jnp.int32(0))

        def wq_copy(src, slot):
            return pltpu.make_async_remote_copy(
                src_ref=src, dst_ref=wqbuf.at[slot],
                send_sem=wqs.at[slot], recv_sem=wqr.at[slot],
                device_id=(right,), device_id_type=pl.DeviceIdType.MESH,
            )

        def wo_copy(src, slot):
            return pltpu.make_async_remote_copy(
                src_ref=src, dst_ref=wobuf.at[slot],
                send_sem=wos.at[slot], recv_sem=wor.at[slot],
                device_id=(left,), device_id_type=pl.DeviceIdType.MESH,
            )

        barrier = pltpu.get_barrier_semaphore()
        for nbr in (left, right):
            pl.semaphore_signal(
                barrier, inc=1,
                device_id=(nbr,), device_id_type=pl.DeviceIdType.MESH,
            )
        pl.semaphore_wait(barrier, 2)

        cq0 = wq_copy(wq_ref, 0)
        cq0.start()
        co0 = wo_copy(wo_ref, 0)
        co0.start()

        for j, blk in enumerate(_PBLK):
            xperm[j * BLK:(j + 1) * BLK, :] = (
                x_ref[0, blk * BLK:(blk + 1) * BLK, :])

        wfull[0:DM, :] = wo_ref[:, :]
        compute_block(wq_ref, my_pos, 0)

        for t in range(1, N_DEV):
            src = (my_pos + N_DEV - t) % N_DEV
            for c in kv_copies(src, 0, 0):
                c.start()
            wq_in = wq_copy(wq_ref, t - 1)
            wq_in.wait()
            wo_in = wo_copy(wo_ref, t - 1)
            wo_in.wait()
            if t < N_DEV - 1:
                cq = wq_copy(wqbuf.at[t - 1], t)
                cq.start()
                co = wo_copy(wobuf.at[t - 1], t)
                co.start()
            wfull[(N_DEV - t) * DM:(N_DEV - t + 1) * DM, :] = wobuf[t - 1]
            compute_block(wqbuf.at[t - 1], src, t, prefetched=True)

        acc = lax.dot_general(
            ctxall[:, :], wfull[:, :], (((1,), (0,)), ((), ())),
            preferred_element_type=jnp.float32,
        )

        for j, blk in enumerate(_PBLK):
            out_ref[0, blk * BLK:(blk + 1) * BLK, :] = (
                acc[j * BLK:(j + 1) * BLK, :])

    out = pl.pallas_call(
        body,
        out_shape=jax.ShapeDtypeStruct((1, SQ, DM), jnp.float32),
        in_specs=[
            pl.BlockSpec(memory_space=pltpu.VMEM),
            pl.BlockSpec(memory_space=pltpu.VMEM),
            pl.BlockSpec(memory_space=pl.ANY),
            pl.BlockSpec(memory_space=pl.ANY),
            pl.BlockSpec(memory_space=pltpu.VMEM),
        ],
        out_specs=pl.BlockSpec(memory_space=pltpu.VMEM),
        scratch_shapes=[
            pltpu.VMEM((3, DM, DM), jnp.bfloat16),
            pltpu.VMEM((3, DM, DM), jnp.bfloat16),
            pltpu.VMEM((SQ, DM), jnp.bfloat16),
            pltpu.VMEM((SQ, DM), jnp.bfloat16),
            pltpu.VMEM((SQ, N_DEV * DM), jnp.bfloat16),
            pltpu.VMEM((N_DEV * DM, DM), jnp.bfloat16),
            pltpu.VMEM((2, SKV, DH), jnp.float32),
            pltpu.VMEM((2, SKV, DH), jnp.float32),
            pltpu.SemaphoreType.DMA((2, NBLK)),
            pltpu.SemaphoreType.DMA((2, NBLK)),
            pltpu.SemaphoreType.DMA((3,)),
            pltpu.SemaphoreType.DMA((3,)),
            pltpu.SemaphoreType.DMA((3,)),
            pltpu.SemaphoreType.DMA((3,)),
        ],
        compiler_params=pltpu.CompilerParams(
            collective_id=0, vmem_limit_bytes=100 * 1024 * 1024,
        ),
    )(x_b, Wq_b, K_ext, V_ext, Wo_b)
    return out
